# Initial kernel scaffold; baseline (speedup 1.0000x reference)
import functools

import jax
import jax.numpy as jnp
from jax import lax
from jax.experimental import pallas as pl
from jax.experimental.pallas import tpu as pltpu
from jax.experimental.pallas import tpu_sc as plsc

N_NODES = 10000
N_EDGES = 320000
D = 128

NC = 2
NS = 16
CHUNK = 128
N_CHUNKS = N_EDGES // CHUNK
CHUNKS_PER_TILE = N_CHUNKS // (NC * NS)
EXTRA_CHUNKS = N_CHUNKS - CHUNKS_PER_TILE * NC * NS
ROWS_PER_TILE = N_NODES // NS


def _sc_aggregate(h, src, dst, zrows):
    mesh = plsc.VectorSubcoreMesh(
        core_axis_name="c", subcore_axis_name="s", num_cores=NC, num_subcores=NS
    )

    @functools.partial(
        pl.kernel,
        out_type=jax.ShapeDtypeStruct((NC * N_NODES, D), jnp.float32),
        mesh=mesh,
        scratch_types=[
            pltpu.VMEM_SHARED((N_NODES, D), jnp.float32),
            pltpu.VMEM((CHUNK,), jnp.int32),
            pltpu.VMEM((CHUNK,), jnp.int32),
            pltpu.VMEM((CHUNK, D), jnp.float32),
            pltpu.SemaphoreType.DMA,
        ],
    )
    def agg(h_hbm, src_hbm, dst_hbm, z_hbm, out_hbm, acc, src_v, dst_v, rows_v, sem):
        cid = lax.axis_index("c")
        sid = lax.axis_index("s")
        wid = cid * NS + sid

        pltpu.sync_copy(z_hbm, acc.at[pl.ds(sid * ROWS_PER_TILE, ROWS_PER_TILE)])
        plsc.subcore_barrier()

        def process(chunk_id):
            off = chunk_id * CHUNK
            pltpu.sync_copy(src_hbm.at[pl.ds(off, CHUNK)], src_v)
            pltpu.sync_copy(dst_hbm.at[pl.ds(off, CHUNK)], dst_v)
            pltpu.async_copy(h_hbm.at[src_v], rows_v, sem).wait()
            pltpu.sync_copy(rows_v, acc.at[dst_v], add=True)

        def body(j, carry):
            process(wid * CHUNKS_PER_TILE + j)
            return carry

        lax.fori_loop(0, CHUNKS_PER_TILE, body, 0)

        @pl.when(wid < EXTRA_CHUNKS)
        def _():
            process(CHUNKS_PER_TILE * NC * NS + wid)

        plsc.subcore_barrier()
        pltpu.sync_copy(
            acc.at[pl.ds(sid * ROWS_PER_TILE, ROWS_PER_TILE)],
            out_hbm.at[pl.ds(cid * N_NODES + sid * ROWS_PER_TILE, ROWS_PER_TILE)],
        )

    return agg(h, src, dst, zrows)


_BN = 1000


def _tc_stage(p, W, b, Wn, relu):
    nb = N_NODES // _BN

    def body(*refs):
        if W is not None and Wn is not None:
            p0, p1, w, bb, wn, o = refs
        elif W is not None:
            p0, p1, w, bb, o = refs
        else:
            p0, p1, bb, o = refs
        s = p0[...] + p1[...]
        if W is not None:
            y = jnp.dot(s, w[...], preferred_element_type=jnp.float32) + bb[...]
        else:
            y = s + bb[...]
        if relu:
            y = jnp.maximum(y, 0.0)
        if Wn is not None:
            y = jnp.dot(y, wn[...], preferred_element_type=jnp.float32)
        o[...] = y

    in_specs = [
        pl.BlockSpec((_BN, D), lambda i: (i, 0)),
        pl.BlockSpec((_BN, D), lambda i: (i + nb, 0)),
    ]
    args = [p, p]
    if W is not None:
        in_specs.append(pl.BlockSpec((D, D), lambda i: (0, 0)))
        args.append(W)
    in_specs.append(pl.BlockSpec((1, D), lambda i: (0, 0)))
    args.append(b.reshape(1, D))
    if Wn is not None:
        in_specs.append(pl.BlockSpec((D, D), lambda i: (0, 0)))
        args.append(Wn)

    return pl.pallas_call(
        body,
        grid=(nb,),
        in_specs=in_specs,
        out_specs=pl.BlockSpec((_BN, D), lambda i: (i, 0)),
        out_shape=jax.ShapeDtypeStruct((N_NODES, D), jnp.float32),
    )(*args)


def kernel(x, edge_index, W1, b1, W2, b2, W3, b3):
    src = edge_index[0]
    dst = edge_index[1]
    zrows = jnp.zeros((ROWS_PER_TILE, D), jnp.float32)
    W3p = jnp.zeros((D, D), jnp.float32).at[:, : W3.shape[1]].set(W3)
    b3p = jnp.zeros((D,), jnp.float32).at[: b3.shape[0]].set(b3)

    p = _sc_aggregate(x, src, dst, zrows)
    h1 = _tc_stage(p, W1, b1, None, relu=True)
    q = _sc_aggregate(h1, src, dst, zrows)
    t = _tc_stage(q, W2, b2, W3p, relu=True)
    r = _sc_aggregate(t, src, dst, zrows)
    out_full = _tc_stage(r, None, b3p, None, relu=False)
    return out_full[:, : W3.shape[1]]


# speedup vs baseline: 5.5665x; 5.5665x over previous
import functools

import jax
import jax.numpy as jnp
from jax import lax
from jax.experimental import pallas as pl
from jax.experimental.pallas import tpu as pltpu
from jax.experimental.pallas import tpu_sc as plsc

N_NODES = 10000
N_EDGES = 320000
D = 128

NC = 2
NS = 16
CHUNK = 128
N_CHUNKS = N_EDGES // CHUNK
CHUNKS_PER_TILE = N_CHUNKS // (NC * NS)
EXTRA_CHUNKS = N_CHUNKS - CHUNKS_PER_TILE * NC * NS
ROWS_PER_TILE = 624
ROWS_REM = N_NODES - NS * ROWS_PER_TILE


def _sc_aggregate(h, src, dst, zrows):
    mesh = plsc.VectorSubcoreMesh(
        core_axis_name="c", subcore_axis_name="s", num_cores=NC, num_subcores=NS
    )

    @functools.partial(
        pl.kernel,
        out_type=jax.ShapeDtypeStruct((NC * N_NODES, D), jnp.float32),
        mesh=mesh,
        scratch_types=[
            pltpu.VMEM_SHARED((N_NODES, D), jnp.float32),
            pltpu.VMEM((CHUNK,), jnp.int32),
            pltpu.VMEM((CHUNK,), jnp.int32),
            pltpu.VMEM((CHUNK, D), jnp.float32),
            pltpu.SemaphoreType.DMA,
        ],
    )
    def agg(h_hbm, src_hbm, dst_hbm, z_hbm, out_hbm, acc, src_v, dst_v, rows_v, sem):
        cid = lax.axis_index("c")
        sid = lax.axis_index("s")
        wid = cid * NS + sid

        pltpu.sync_copy(
            z_hbm.at[pl.ds(0, ROWS_PER_TILE)],
            acc.at[pl.ds(sid * ROWS_PER_TILE, ROWS_PER_TILE)],
        )

        @pl.when(sid == 0)
        def _():
            pltpu.sync_copy(
                z_hbm.at[pl.ds(0, ROWS_REM)],
                acc.at[pl.ds(NS * ROWS_PER_TILE, ROWS_REM)],
            )

        plsc.subcore_barrier()

        def process(chunk_id):
            off = chunk_id * CHUNK
            pltpu.sync_copy(src_hbm.at[pl.ds(off, CHUNK)], src_v)
            pltpu.sync_copy(dst_hbm.at[pl.ds(off, CHUNK)], dst_v)
            pltpu.async_copy(h_hbm.at[src_v], rows_v, sem).wait()
            pltpu.sync_copy(rows_v, acc.at[dst_v], add=True)

        def body(j, carry):
            process(wid * CHUNKS_PER_TILE + j)
            return carry

        lax.fori_loop(0, CHUNKS_PER_TILE, body, 0)

        @pl.when(wid < EXTRA_CHUNKS)
        def _():
            process(CHUNKS_PER_TILE * NC * NS + wid)

        plsc.subcore_barrier()
        pltpu.sync_copy(
            acc.at[pl.ds(sid * ROWS_PER_TILE, ROWS_PER_TILE)],
            out_hbm.at[pl.ds(cid * N_NODES + sid * ROWS_PER_TILE, ROWS_PER_TILE)],
        )

        @pl.when(sid == 0)
        def _():
            pltpu.sync_copy(
                acc.at[pl.ds(NS * ROWS_PER_TILE, ROWS_REM)],
                out_hbm.at[pl.ds(cid * N_NODES + NS * ROWS_PER_TILE, ROWS_REM)],
            )

    return agg(h, src, dst, zrows)


_BN = 1000


def _tc_stage(p, W, b, Wn, relu):
    nb = N_NODES // _BN

    def body(*refs):
        if W is not None and Wn is not None:
            p0, p1, w, bb, wn, o = refs
        elif W is not None:
            p0, p1, w, bb, o = refs
        else:
            p0, p1, bb, o = refs
        s = p0[...] + p1[...]
        if W is not None:
            y = jnp.dot(s, w[...], preferred_element_type=jnp.float32) + bb[...]
        else:
            y = s + bb[...]
        if relu:
            y = jnp.maximum(y, 0.0)
        if Wn is not None:
            y = jnp.dot(y, wn[...], preferred_element_type=jnp.float32)
        o[...] = y

    in_specs = [
        pl.BlockSpec((_BN, D), lambda i: (i, 0)),
        pl.BlockSpec((_BN, D), lambda i: (i + nb, 0)),
    ]
    args = [p, p]
    if W is not None:
        in_specs.append(pl.BlockSpec((D, D), lambda i: (0, 0)))
        args.append(W)
    in_specs.append(pl.BlockSpec((1, D), lambda i: (0, 0)))
    args.append(b.reshape(1, D))
    if Wn is not None:
        in_specs.append(pl.BlockSpec((D, D), lambda i: (0, 0)))
        args.append(Wn)

    return pl.pallas_call(
        body,
        grid=(nb,),
        in_specs=in_specs,
        out_specs=pl.BlockSpec((_BN, D), lambda i: (i, 0)),
        out_shape=jax.ShapeDtypeStruct((N_NODES, D), jnp.float32),
    )(*args)


def kernel(x, edge_index, W1, b1, W2, b2, W3, b3):
    src = edge_index[0]
    dst = edge_index[1]
    zrows = jnp.zeros((ROWS_PER_TILE, D), jnp.float32)
    W3p = jnp.zeros((D, D), jnp.float32).at[:, : W3.shape[1]].set(W3)
    b3p = jnp.zeros((D,), jnp.float32).at[: b3.shape[0]].set(b3)

    p = _sc_aggregate(x, src, dst, zrows)
    h1 = _tc_stage(p, W1, b1, None, relu=True)
    q = _sc_aggregate(h1, src, dst, zrows)
    t = _tc_stage(q, W2, b2, W3p, relu=True)
    r = _sc_aggregate(t, src, dst, zrows)
    out_full = _tc_stage(r, None, b3p, None, relu=False)
    return out_full[:, : W3.shape[1]]


# speedup vs baseline: 14.1443x; 2.5410x over previous
import functools

import jax
import jax.numpy as jnp
from jax import lax
from jax.experimental import pallas as pl
from jax.experimental.pallas import tpu as pltpu
from jax.experimental.pallas import tpu_sc as plsc

N_NODES = 10000
N_EDGES = 320000
D = 128
D3 = 64

NC = 2
NS = 16
CHUNK = 80
N_CHUNKS = N_EDGES // CHUNK
CHUNKS_PER_TILE = N_CHUNKS // (NC * NS)
NBUF = 4
NSLOT = 8
ROWS_PER_TILE = 624
ROWS_REM = N_NODES - NS * ROWS_PER_TILE


def _sc_aggregate(h, src1d, dst1d, zrows, d):
    mesh = plsc.VectorSubcoreMesh(
        core_axis_name="c", subcore_axis_name="s", num_cores=NC, num_subcores=NS
    )
    n = CHUNKS_PER_TILE

    @functools.partial(
        pl.kernel,
        out_type=jax.ShapeDtypeStruct((NC * N_NODES, d), jnp.float32),
        mesh=mesh,
        scratch_types=[
            pltpu.VMEM_SHARED((N_NODES, d), jnp.float32),
            pltpu.VMEM((NSLOT, CHUNK), jnp.int32),
            pltpu.VMEM((NSLOT, CHUNK), jnp.int32),
            pltpu.VMEM((NBUF, CHUNK, d), jnp.float32),
            [pltpu.SemaphoreType.DMA] * NSLOT,
            [pltpu.SemaphoreType.DMA] * NBUF,
            pltpu.SemaphoreType.DMA,
        ],
        compiler_params=pltpu.CompilerParams(use_tc_tiling_on_sc=(d == D)),
    )
    def agg(h_hbm, src_hbm, dst_hbm, z_hbm, out_hbm, acc, src_v, dst_v, rows,
            isem, gsem, zsem):
        cid = lax.axis_index("c")
        sid = lax.axis_index("s")
        wid = cid * NS + sid
        base = wid * (n * CHUNK)

        pltpu.async_copy(
            z_hbm.at[pl.ds(0, ROWS_PER_TILE)],
            acc.at[pl.ds(sid * ROWS_PER_TILE, ROWS_PER_TILE)],
            zsem,
        )

        @pl.when(sid == 0)
        def _():
            pltpu.async_copy(
                z_hbm.at[pl.ds(0, ROWS_REM)],
                acc.at[pl.ds(NS * ROWS_PER_TILE, ROWS_REM)],
                zsem,
            )

        def start_i(c, sl):
            off = base + c * CHUNK
            pltpu.async_copy(src_hbm.at[pl.ds(off, CHUNK)], src_v.at[sl], isem[sl])
            pltpu.async_copy(dst_hbm.at[pl.ds(off, CHUNK)], dst_v.at[sl], isem[sl])

        def wait_i(c, sl):
            off = base + c * CHUNK
            pltpu.make_async_copy(src_hbm.at[pl.ds(off, CHUNK)], src_v.at[sl], isem[sl]).wait()
            pltpu.make_async_copy(dst_hbm.at[pl.ds(off, CHUNK)], dst_v.at[sl], isem[sl]).wait()

        def start_g(sl, r):
            pltpu.async_copy(h_hbm.at[src_v.at[sl]], rows.at[r], gsem[r])

        def wait_g(sl, r):
            pltpu.make_async_copy(h_hbm.at[src_v.at[sl]], rows.at[r], gsem[r]).wait()

        def do_s(sl, r):
            pltpu.sync_copy(rows.at[r], acc.at[dst_v.at[sl]], add=True)

        def pipe_iter(c, k, has_g_next=True, has_i_next=True):
            i3, i6 = (k + 3) % NSLOT, (k + 6) % NSLOT
            r0, r3 = k % NBUF, (k + 3) % NBUF
            if has_g_next:
                wait_i(c + 3, i3)
                start_g(i3, r3)
            if has_i_next:
                start_i(c + 6, i6)
            wait_g(k, r0)
            do_s(k, r0)

        for c0 in range(6):
            start_i(c0, c0)
        for c0 in range(3):
            wait_i(c0, c0)
            start_g(c0, c0)

        pltpu.make_async_copy(
            z_hbm.at[pl.ds(0, ROWS_PER_TILE)],
            acc.at[pl.ds(sid * ROWS_PER_TILE, ROWS_PER_TILE)],
            zsem,
        ).wait()

        @pl.when(sid == 0)
        def _():
            pltpu.make_async_copy(
                z_hbm.at[pl.ds(0, ROWS_REM)],
                acc.at[pl.ds(NS * ROWS_PER_TILE, ROWS_REM)],
                zsem,
            ).wait()

        plsc.subcore_barrier()

        pipe_iter(0, 0)
        pipe_iter(1, 1)

        def block(i, carry):
            cb = 2 + i * NSLOT
            for kk in range(NSLOT):
                pipe_iter(cb + kk, (2 + kk) % NSLOT)
            return carry

        n_blocks = (n - 8) // NSLOT
        lax.fori_loop(0, n_blocks, block, 0)

        for c in range(2 + NSLOT * n_blocks, n):
            pipe_iter(c, c % NSLOT,
                      has_g_next=(c + 3 <= n - 1), has_i_next=(c + 6 <= n - 1))

        plsc.subcore_barrier()
        pltpu.sync_copy(
            acc.at[pl.ds(sid * ROWS_PER_TILE, ROWS_PER_TILE)],
            out_hbm.at[pl.ds(cid * N_NODES + sid * ROWS_PER_TILE, ROWS_PER_TILE)],
        )

        @pl.when(sid == 0)
        def _():
            pltpu.sync_copy(
                acc.at[pl.ds(NS * ROWS_PER_TILE, ROWS_REM)],
                out_hbm.at[pl.ds(cid * N_NODES + NS * ROWS_PER_TILE, ROWS_REM)],
            )

    return agg(h, src1d, dst1d, zrows)


_BN = 1000


def _tc_stage(p, W, b, Wn, relu):
    nb = N_NODES // _BN
    dp = p.shape[-1]
    dmid = W.shape[-1] if W is not None else dp
    dout = Wn.shape[-1] if Wn is not None else dmid

    def body(*refs):
        if W is not None and Wn is not None:
            p0, p1, w, bb, wn, o = refs
        elif W is not None:
            p0, p1, w, bb, o = refs
        else:
            p0, p1, bb, o = refs
        s = p0[...] + p1[...]
        if W is not None:
            y = jnp.dot(s, w[...], preferred_element_type=jnp.float32) + bb[...]
        else:
            y = s + bb[...]
        if relu:
            y = jnp.maximum(y, 0.0)
        if Wn is not None:
            y = jnp.dot(y, wn[...], preferred_element_type=jnp.float32)
        o[...] = y

    in_specs = [
        pl.BlockSpec((_BN, dp), lambda i: (i, 0)),
        pl.BlockSpec((_BN, dp), lambda i: (i + nb, 0)),
    ]
    args = [p, p]
    if W is not None:
        in_specs.append(pl.BlockSpec((dp, dmid), lambda i: (0, 0)))
        args.append(W)
    in_specs.append(pl.BlockSpec((1, dmid), lambda i: (0, 0)))
    args.append(b.reshape(1, dmid))
    if Wn is not None:
        in_specs.append(pl.BlockSpec((dmid, dout), lambda i: (0, 0)))
        args.append(Wn)

    return pl.pallas_call(
        body,
        grid=(nb,),
        in_specs=in_specs,
        out_specs=pl.BlockSpec((_BN, dout), lambda i: (i, 0)),
        out_shape=jax.ShapeDtypeStruct((N_NODES, dout), jnp.float32),
    )(*args)


def kernel(x, edge_index, W1, b1, W2, b2, W3, b3):
    src = edge_index[0]
    dst = edge_index[1]
    zrows = jnp.zeros((ROWS_PER_TILE, D), jnp.float32)
    zrows3 = jnp.zeros((ROWS_PER_TILE, D3), jnp.float32)
    W3p = jnp.zeros((D, D3), jnp.float32).at[:, : W3.shape[1]].set(W3)
    b3p = jnp.zeros((D3,), jnp.float32).at[: b3.shape[0]].set(b3)

    p = _sc_aggregate(x, src, dst, zrows, D)
    h1 = _tc_stage(p, W1, b1, None, relu=True)
    q = _sc_aggregate(h1, src, dst, zrows, D)
    t = _tc_stage(q, W2, b2, W3p, relu=True)
    r = _sc_aggregate(t, src, dst, zrows3, D3)
    out_full = _tc_stage(r, None, b3p, None, relu=False)
    return out_full[:, : W3.shape[1]]


# speedup vs baseline: 15.2488x; 1.0781x over previous
import functools

import jax
import jax.numpy as jnp
from jax import lax
from jax.experimental import pallas as pl
from jax.experimental.pallas import tpu as pltpu
from jax.experimental.pallas import tpu_sc as plsc

N_NODES = 10000
N_EDGES = 320000
D = 128
D3 = 64

NC = 2
NS = 16
CHUNK = 80
N_CHUNKS = N_EDGES // CHUNK
CHUNKS_PER_TILE = N_CHUNKS // (NC * NS)
NBUF = 4
NSLOT = 8
ROWS_PER_TILE = 624
ROWS_REM = N_NODES - NS * ROWS_PER_TILE


def _sc_aggregate(h, src1d, dst1d, d):
    mesh = plsc.VectorSubcoreMesh(
        core_axis_name="c", subcore_axis_name="s", num_cores=NC, num_subcores=NS
    )
    n = CHUNKS_PER_TILE

    @functools.partial(
        pl.kernel,
        out_type=jax.ShapeDtypeStruct((NC * N_NODES, d), jnp.float32),
        mesh=mesh,
        scratch_types=[
            pltpu.VMEM_SHARED((N_NODES, d), jnp.float32),
            pltpu.VMEM((NSLOT, CHUNK), jnp.int32),
            pltpu.VMEM((NSLOT, CHUNK), jnp.int32),
            pltpu.VMEM((NBUF, CHUNK, d), jnp.float32),
            [pltpu.SemaphoreType.DMA] * NSLOT,
            [pltpu.SemaphoreType.DMA] * NBUF,
            pltpu.SemaphoreType.DMA,
        ],
        compiler_params=pltpu.CompilerParams(use_tc_tiling_on_sc=(d == D)),
    )
    def agg(h_hbm, src_hbm, dst_hbm, out_hbm, acc, src_v, dst_v, rows,
            isem, gsem, zsem):
        cid = lax.axis_index("c")
        sid = lax.axis_index("s")
        wid = cid * NS + sid
        base = wid * (n * CHUNK)

        ztile = rows.at[NBUF - 1]

        def zfill(i, carry):
            for j in range(d // 16):
                rows[NBUF - 1, i, pl.ds(j * 16, 16)] = jnp.zeros((16,), jnp.float32)
            return carry

        lax.fori_loop(0, CHUNK, zfill, 0)

        nzc = ROWS_PER_TILE // CHUNK
        zrem = ROWS_PER_TILE - nzc * CHUNK
        for j in range(nzc):
            pltpu.async_copy(
                ztile, acc.at[pl.ds(sid * ROWS_PER_TILE + j * CHUNK, CHUNK)], zsem)
        pltpu.async_copy(
            ztile.at[pl.ds(0, zrem)],
            acc.at[pl.ds(sid * ROWS_PER_TILE + nzc * CHUNK, zrem)], zsem)

        @pl.when(sid == 0)
        def _():
            pltpu.async_copy(
                ztile.at[pl.ds(0, ROWS_REM)],
                acc.at[pl.ds(NS * ROWS_PER_TILE, ROWS_REM)], zsem)

        def start_i(c, sl):
            off = base + c * CHUNK
            pltpu.async_copy(src_hbm.at[pl.ds(off, CHUNK)], src_v.at[sl], isem[sl])
            pltpu.async_copy(dst_hbm.at[pl.ds(off, CHUNK)], dst_v.at[sl], isem[sl])

        def wait_i(c, sl):
            off = base + c * CHUNK
            pltpu.make_async_copy(src_hbm.at[pl.ds(off, CHUNK)], src_v.at[sl], isem[sl]).wait()
            pltpu.make_async_copy(dst_hbm.at[pl.ds(off, CHUNK)], dst_v.at[sl], isem[sl]).wait()

        def start_g(sl, r):
            pltpu.async_copy(h_hbm.at[src_v.at[sl]], rows.at[r], gsem[r])

        def wait_g(sl, r):
            pltpu.make_async_copy(h_hbm.at[src_v.at[sl]], rows.at[r], gsem[r]).wait()

        def do_s(sl, r):
            pltpu.sync_copy(rows.at[r], acc.at[dst_v.at[sl]], add=True)

        def pipe_iter(c, k, has_g_next=True, has_i_next=True):
            i3, i6 = (k + 3) % NSLOT, (k + 6) % NSLOT
            r0, r3 = k % NBUF, (k + 3) % NBUF
            if has_g_next:
                wait_i(c + 3, i3)
                start_g(i3, r3)
            if has_i_next:
                start_i(c + 6, i6)
            wait_g(k, r0)
            do_s(k, r0)

        for c0 in range(6):
            start_i(c0, c0)
        for c0 in range(3):
            wait_i(c0, c0)
            start_g(c0, c0)

        for j in range(nzc):
            pltpu.make_async_copy(
                ztile, acc.at[pl.ds(sid * ROWS_PER_TILE + j * CHUNK, CHUNK)], zsem).wait()
        pltpu.make_async_copy(
            ztile.at[pl.ds(0, zrem)],
            acc.at[pl.ds(sid * ROWS_PER_TILE + nzc * CHUNK, zrem)], zsem).wait()

        @pl.when(sid == 0)
        def _():
            pltpu.make_async_copy(
                ztile.at[pl.ds(0, ROWS_REM)],
                acc.at[pl.ds(NS * ROWS_PER_TILE, ROWS_REM)], zsem).wait()

        plsc.subcore_barrier()

        pipe_iter(0, 0)
        pipe_iter(1, 1)

        def block(i, carry):
            cb = 2 + i * NSLOT
            for kk in range(NSLOT):
                pipe_iter(cb + kk, (2 + kk) % NSLOT)
            return carry

        n_blocks = (n - 8) // NSLOT
        lax.fori_loop(0, n_blocks, block, 0)

        for c in range(2 + NSLOT * n_blocks, n):
            pipe_iter(c, c % NSLOT,
                      has_g_next=(c + 3 <= n - 1), has_i_next=(c + 6 <= n - 1))

        plsc.subcore_barrier()
        pltpu.sync_copy(
            acc.at[pl.ds(sid * ROWS_PER_TILE, ROWS_PER_TILE)],
            out_hbm.at[pl.ds(cid * N_NODES + sid * ROWS_PER_TILE, ROWS_PER_TILE)],
        )

        @pl.when(sid == 0)
        def _():
            pltpu.sync_copy(
                acc.at[pl.ds(NS * ROWS_PER_TILE, ROWS_REM)],
                out_hbm.at[pl.ds(cid * N_NODES + NS * ROWS_PER_TILE, ROWS_REM)],
            )

    return agg(h, src1d, dst1d)


_BN = 2000


def _tc_stage(p, W, b, Wn, relu):
    nb = N_NODES // _BN
    dp = p.shape[-1]
    dmid = W.shape[-1] if W is not None else dp
    dout = Wn.shape[-1] if Wn is not None else dmid

    def body(*refs):
        if W is not None and Wn is not None:
            p0, p1, w, bb, wn, o = refs
        elif W is not None:
            p0, p1, w, bb, o = refs
        else:
            p0, p1, bb, o = refs
        s = p0[...] + p1[...]
        if W is not None:
            y = jnp.dot(s, w[...], preferred_element_type=jnp.float32) + bb[...]
        else:
            y = s + bb[...]
        if relu:
            y = jnp.maximum(y, 0.0)
        if Wn is not None:
            y = jnp.dot(y, wn[...], preferred_element_type=jnp.float32)
        o[...] = y

    in_specs = [
        pl.BlockSpec((_BN, dp), lambda i: (i, 0)),
        pl.BlockSpec((_BN, dp), lambda i: (i + nb, 0)),
    ]
    args = [p, p]
    if W is not None:
        in_specs.append(pl.BlockSpec((dp, dmid), lambda i: (0, 0)))
        args.append(W)
    in_specs.append(pl.BlockSpec((1, dmid), lambda i: (0, 0)))
    args.append(b.reshape(1, dmid))
    if Wn is not None:
        in_specs.append(pl.BlockSpec((dmid, dout), lambda i: (0, 0)))
        args.append(Wn)

    return pl.pallas_call(
        body,
        grid=(nb,),
        in_specs=in_specs,
        out_specs=pl.BlockSpec((_BN, dout), lambda i: (i, 0)),
        out_shape=jax.ShapeDtypeStruct((N_NODES, dout), jnp.float32),
    )(*args)


def kernel(x, edge_index, W1, b1, W2, b2, W3, b3):
    src = edge_index[0]
    dst = edge_index[1]
    W3p = jnp.zeros((D, D3), jnp.float32).at[:, : W3.shape[1]].set(W3)
    b3p = jnp.zeros((D3,), jnp.float32).at[: b3.shape[0]].set(b3)

    p = _sc_aggregate(x, src, dst, D)
    h1 = _tc_stage(p, W1, b1, None, relu=True)
    q = _sc_aggregate(h1, src, dst, D)
    t = _tc_stage(q, W2, b2, W3p, relu=True)
    r = _sc_aggregate(t, src, dst, D3)
    out_full = _tc_stage(r, None, b3p, None, relu=False)
    return out_full[:, : W3.shape[1]]


# speedup vs baseline: 15.2496x; 1.0001x over previous
import functools

import jax
import jax.numpy as jnp
from jax import lax
from jax.experimental import pallas as pl
from jax.experimental.pallas import tpu as pltpu
from jax.experimental.pallas import tpu_sc as plsc

N_NODES = 10000
N_EDGES = 320000
D = 128
D3 = 64

NC = 2
NS = 16
CHUNK = 80
N_CHUNKS = N_EDGES // CHUNK
CHUNKS_PER_TILE = N_CHUNKS // (NC * NS)
NBUF = 4
NSLOT = 8
ROWS_PER_TILE = 624
ROWS_REM = N_NODES - NS * ROWS_PER_TILE


def _sc_aggregate(h, src1d, dst1d, d):
    mesh = plsc.VectorSubcoreMesh(
        core_axis_name="c", subcore_axis_name="s", num_cores=NC, num_subcores=NS
    )
    n = CHUNKS_PER_TILE

    @functools.partial(
        pl.kernel,
        out_type=jax.ShapeDtypeStruct((NC * N_NODES, d), jnp.float32),
        mesh=mesh,
        scratch_types=[
            pltpu.VMEM_SHARED((N_NODES, d), jnp.float32),
            pltpu.VMEM((NSLOT, CHUNK), jnp.int32),
            pltpu.VMEM((NSLOT, CHUNK), jnp.int32),
            pltpu.VMEM((NBUF, CHUNK, d), jnp.float32),
            [pltpu.SemaphoreType.DMA] * NSLOT,
            [pltpu.SemaphoreType.DMA] * NBUF,
            [pltpu.SemaphoreType.DMA] * NBUF,
            pltpu.SemaphoreType.DMA,
        ],
        compiler_params=pltpu.CompilerParams(use_tc_tiling_on_sc=(d == D)),
    )
    def agg(h_hbm, src_hbm, dst_hbm, out_hbm, acc, src_v, dst_v, rows,
            isem, gsem, ssem, zsem):
        cid = lax.axis_index("c")
        sid = lax.axis_index("s")
        wid = cid * NS + sid
        base = wid * (n * CHUNK)

        ztile = rows.at[NBUF - 1]

        def zfill(i, carry):
            for j in range(d // 16):
                rows[NBUF - 1, i, pl.ds(j * 16, 16)] = jnp.zeros((16,), jnp.float32)
            return carry

        lax.fori_loop(0, CHUNK, zfill, 0)

        nzc = ROWS_PER_TILE // CHUNK
        zrem = ROWS_PER_TILE - nzc * CHUNK
        for j in range(nzc):
            pltpu.async_copy(
                ztile, acc.at[pl.ds(sid * ROWS_PER_TILE + j * CHUNK, CHUNK)], zsem)
        pltpu.async_copy(
            ztile.at[pl.ds(0, zrem)],
            acc.at[pl.ds(sid * ROWS_PER_TILE + nzc * CHUNK, zrem)], zsem)

        @pl.when(sid == 0)
        def _():
            pltpu.async_copy(
                ztile.at[pl.ds(0, ROWS_REM)],
                acc.at[pl.ds(NS * ROWS_PER_TILE, ROWS_REM)], zsem)

        def start_i(c, sl):
            off = base + c * CHUNK
            pltpu.async_copy(src_hbm.at[pl.ds(off, CHUNK)], src_v.at[sl], isem[sl])
            pltpu.async_copy(dst_hbm.at[pl.ds(off, CHUNK)], dst_v.at[sl], isem[sl])

        def wait_i(c, sl):
            off = base + c * CHUNK
            pltpu.make_async_copy(src_hbm.at[pl.ds(off, CHUNK)], src_v.at[sl], isem[sl]).wait()
            pltpu.make_async_copy(dst_hbm.at[pl.ds(off, CHUNK)], dst_v.at[sl], isem[sl]).wait()

        def start_g(sl, r):
            pltpu.async_copy(h_hbm.at[src_v.at[sl]], rows.at[r], gsem[r])

        def wait_g(sl, r):
            pltpu.make_async_copy(h_hbm.at[src_v.at[sl]], rows.at[r], gsem[r]).wait()

        def start_s(sl, r):
            pltpu.async_copy(rows.at[r], acc.at[dst_v.at[sl]], ssem[r], add=True)

        def wait_s(sl, r):
            pltpu.make_async_copy(rows.at[r], acc.at[dst_v.at[sl]], ssem[r]).wait()

        def pipe_iter(c, k, has_s_prev=True, has_g_next=True, has_i_next=True):
            i3, i6 = (k + 3) % NSLOT, (k + 6) % NSLOT
            r0, r3 = k % NBUF, (k + 3) % NBUF
            if has_g_next:
                wait_i(c + 3, i3)
                if has_s_prev:
                    wait_s((k + 7) % NSLOT, r3)
                start_g(i3, r3)
            if has_i_next:
                start_i(c + 6, i6)
            wait_g(k, r0)
            start_s(k, r0)

        for c0 in range(6):
            start_i(c0, c0)
        for c0 in range(3):
            wait_i(c0, c0)
            start_g(c0, c0)

        for j in range(nzc):
            pltpu.make_async_copy(
                ztile, acc.at[pl.ds(sid * ROWS_PER_TILE + j * CHUNK, CHUNK)], zsem).wait()
        pltpu.make_async_copy(
            ztile.at[pl.ds(0, zrem)],
            acc.at[pl.ds(sid * ROWS_PER_TILE + nzc * CHUNK, zrem)], zsem).wait()

        @pl.when(sid == 0)
        def _():
            pltpu.make_async_copy(
                ztile.at[pl.ds(0, ROWS_REM)],
                acc.at[pl.ds(NS * ROWS_PER_TILE, ROWS_REM)], zsem).wait()

        plsc.subcore_barrier()

        pipe_iter(0, 0, has_s_prev=False)
        pipe_iter(1, 1)

        def block(i, carry):
            cb = 2 + i * NSLOT
            for kk in range(NSLOT):
                pipe_iter(cb + kk, (2 + kk) % NSLOT)
            return carry

        n_blocks = (n - 8) // NSLOT
        lax.fori_loop(0, n_blocks, block, 0)

        for c in range(2 + NSLOT * n_blocks, n):
            pipe_iter(c, c % NSLOT,
                      has_g_next=(c + 3 <= n - 1), has_i_next=(c + 6 <= n - 1))

        for c in range(n - 4, n):
            wait_s(c % NSLOT, c % NBUF)

        plsc.subcore_barrier()
        pltpu.sync_copy(
            acc.at[pl.ds(sid * ROWS_PER_TILE, ROWS_PER_TILE)],
            out_hbm.at[pl.ds(cid * N_NODES + sid * ROWS_PER_TILE, ROWS_PER_TILE)],
        )

        @pl.when(sid == 0)
        def _():
            pltpu.sync_copy(
                acc.at[pl.ds(NS * ROWS_PER_TILE, ROWS_REM)],
                out_hbm.at[pl.ds(cid * N_NODES + NS * ROWS_PER_TILE, ROWS_REM)],
            )

    return agg(h, src1d, dst1d)


_BN = 2000


def _tc_stage(p, W, b, Wn, relu):
    nb = N_NODES // _BN
    dp = p.shape[-1]
    dmid = W.shape[-1] if W is not None else dp
    dout = Wn.shape[-1] if Wn is not None else dmid

    def body(*refs):
        if W is not None and Wn is not None:
            p0, p1, w, bb, wn, o = refs
        elif W is not None:
            p0, p1, w, bb, o = refs
        else:
            p0, p1, bb, o = refs
        s = p0[...] + p1[...]
        if W is not None:
            y = jnp.dot(s, w[...], preferred_element_type=jnp.float32) + bb[...]
        else:
            y = s + bb[...]
        if relu:
            y = jnp.maximum(y, 0.0)
        if Wn is not None:
            y = jnp.dot(y, wn[...], preferred_element_type=jnp.float32)
        o[...] = y

    in_specs = [
        pl.BlockSpec((_BN, dp), lambda i: (i, 0)),
        pl.BlockSpec((_BN, dp), lambda i: (i + nb, 0)),
    ]
    args = [p, p]
    if W is not None:
        in_specs.append(pl.BlockSpec((dp, dmid), lambda i: (0, 0)))
        args.append(W)
    in_specs.append(pl.BlockSpec((1, dmid), lambda i: (0, 0)))
    args.append(b.reshape(1, dmid))
    if Wn is not None:
        in_specs.append(pl.BlockSpec((dmid, dout), lambda i: (0, 0)))
        args.append(Wn)

    return pl.pallas_call(
        body,
        grid=(nb,),
        in_specs=in_specs,
        out_specs=pl.BlockSpec((_BN, dout), lambda i: (i, 0)),
        out_shape=jax.ShapeDtypeStruct((N_NODES, dout), jnp.float32),
    )(*args)


def kernel(x, edge_index, W1, b1, W2, b2, W3, b3):
    src = edge_index[0]
    dst = edge_index[1]
    W3p = jnp.zeros((D, D3), jnp.float32).at[:, : W3.shape[1]].set(W3)
    b3p = jnp.zeros((D3,), jnp.float32).at[: b3.shape[0]].set(b3)

    p = _sc_aggregate(x, src, dst, D)
    h1 = _tc_stage(p, W1, b1, None, relu=True)
    q = _sc_aggregate(h1, src, dst, D)
    t = _tc_stage(q, W2, b2, W3p, relu=True)
    r = _sc_aggregate(t, src, dst, D3)
    out_full = _tc_stage(r, None, b3p, None, relu=False)
    return out_full[:, : W3.shape[1]]
